# baseline (speedup 1.0000x reference)
import functools

import jax
import jax.numpy as jnp
from jax import lax
from jax.experimental import pallas as pl
from jax.experimental.pallas import tpu as pltpu

LEAK_FACTOR = 0.2
NUM_TOPICS = 20
HIDDEN = 10
OUT = 4
BN_EPS = 1e-5
LANE = 128


def _round_up(x, m):
    return (x + m - 1) // m * m


def _fused_kernel(x_ref, w1_ref, b1_ref, gamma_ref, beta_ref, w2_ref, b2_ref,
                  o_ref, h_scr, acc_ref, *, batch, tile_cols):
    phase = pl.program_id(0)
    i = pl.program_id(1)

    @pl.when(phase == 0)
    def _stats_phase():
        x = x_ref[...]
        w1 = w1_ref[...]
        b1 = jnp.transpose(b1_ref[...])
        h = lax.dot_general(w1, x, (((0,), (0,)), ((), ())),
                            preferred_element_type=jnp.float32) + b1
        h = jnp.maximum(h, LEAK_FACTOR * h)
        h_scr[i] = h.astype(h_scr.dtype)

        tail = batch % tile_cols
        if tail != 0:
            is_last = i == pl.num_programs(1) - 1
            lane = lax.broadcasted_iota(jnp.int32, h.shape, 1)
            h = jnp.where(jnp.logical_or(jnp.logical_not(is_last),
                                         lane < tail), h, 0.0)

        @pl.when(i == 0)
        def _():
            acc_ref[...] = jnp.zeros_like(acc_ref)

        acc_ref[0:HIDDEN, 0:1] = acc_ref[0:HIDDEN, 0:1] + \
            jnp.sum(h, axis=1, keepdims=True)
        acc_ref[0:HIDDEN, 1:2] = acc_ref[0:HIDDEN, 1:2] + \
            jnp.sum(h * h, axis=1, keepdims=True)

    @pl.when(phase == 1)
    def _apply_phase():
        sums = acc_ref[0:HIDDEN, 0:1]
        sqs = acc_ref[0:HIDDEN, 1:2]
        mean = sums / batch
        var = jnp.maximum(sqs / batch - mean * mean, 0.0)
        gamma = jnp.transpose(gamma_ref[...])
        beta = jnp.transpose(beta_ref[...])
        scale = gamma * lax.rsqrt(var + BN_EPS)
        shift = beta - mean * scale
        w2 = w2_ref[...]
        w2s = (w2 * scale).astype(jnp.bfloat16)
        b2p = jnp.transpose(b2_ref[...] +
                            lax.dot_general(shift, w2, (((0,), (0,)), ((), ())),
                                            preferred_element_type=jnp.float32))
        h = h_scr[i]
        logits = lax.dot_general(w2s, h, (((0,), (0,)), ((), ())),
                                 preferred_element_type=jnp.float32) + b2p
        m = jnp.max(logits, axis=0, keepdims=True)
        e = jnp.exp(logits - m)
        denom = jnp.sum(e, axis=0, keepdims=True)
        o_ref[...] = (e / denom).astype(o_ref.dtype)


def kernel(noise, w1, b1, gamma, beta, w2, b2, *, block_cols=65536):
    B = noise.shape[0]
    tb = max(LANE, min(_round_up(block_cols, LANE), _round_up(B, LANE)))
    bp = _round_up(B, tb)
    nbt = bp // tb

    xt = jnp.pad(jnp.asarray(noise, jnp.float32), ((0, bp - B), (0, 0))).T

    f32 = jnp.float32
    w1f = jnp.asarray(w1, f32)
    b1f = jnp.asarray(b1, f32).reshape(1, HIDDEN)
    gammaf = jnp.asarray(gamma, f32).reshape(1, HIDDEN)
    betaf = jnp.asarray(beta, f32).reshape(1, HIDDEN)
    w2f = jnp.asarray(w2, f32)
    b2f = jnp.asarray(b2, f32).reshape(1, OUT)

    def whole(shape):
        return pl.BlockSpec(shape, lambda p_, i: tuple(0 for _ in shape))

    out_t = pl.pallas_call(
        functools.partial(_fused_kernel, batch=B, tile_cols=tb),
        out_shape=jax.ShapeDtypeStruct((OUT, bp), jnp.float32),
        grid=(2, nbt),
        in_specs=[
            pl.BlockSpec((NUM_TOPICS, tb), lambda p_, i: (0, i * (1 - p_))),
            whole((NUM_TOPICS, HIDDEN)),
            whole((1, HIDDEN)),
            whole((1, HIDDEN)),
            whole((1, HIDDEN)),
            whole((HIDDEN, OUT)),
            whole((1, OUT)),
        ],
        out_specs=pl.BlockSpec((OUT, tb), lambda p_, i: (0, i * p_)),
        scratch_shapes=[
            pltpu.VMEM((nbt, HIDDEN, tb), jnp.bfloat16),
            pltpu.VMEM((HIDDEN + 6, LANE), jnp.float32),
        ],
        cost_estimate=pl.CostEstimate(
            flops=2 * bp * NUM_TOPICS * HIDDEN + 2 * bp * HIDDEN * OUT
                  + 18 * bp * HIDDEN,
            transcendentals=bp * OUT,
            bytes_accessed=4 * (NUM_TOPICS * bp + OUT * bp + 300)),
        compiler_params=pltpu.CompilerParams(
            dimension_semantics=("arbitrary", "arbitrary"),
            vmem_limit_bytes=64 * 1024 * 1024,
        ),
    )(xt, w1f, b1f, gammaf, betaf, w2f, b2f)

    return out_t.T[:B]


# speedup vs baseline: 1.0816x; 1.0816x over previous
import functools

import jax
import jax.numpy as jnp
from jax import lax
from jax.experimental import pallas as pl
from jax.experimental.pallas import tpu as pltpu

LEAK_FACTOR = 0.2
NUM_TOPICS = 20
HIDDEN = 10
OUT = 4
BN_EPS = 1e-5
LANE = 128


def _round_up(x, m):
    return (x + m - 1) // m * m


def _fused_kernel(x_ref, w1_ref, b1_ref, gamma_ref, beta_ref, w2_ref, b2_ref,
                  o_ref, h_scr, acc_ref, *, batch, tile_cols):
    phase = pl.program_id(0)
    i = pl.program_id(1)

    @pl.when(phase == 0)
    def _stats_phase():
        x = x_ref[...]
        w1 = w1_ref[...]
        b1 = jnp.transpose(b1_ref[...])
        h = lax.dot_general(w1, x, (((0,), (0,)), ((), ())),
                            preferred_element_type=jnp.float32) + b1
        h = jnp.maximum(h, LEAK_FACTOR * h)
        h_scr[i] = h.astype(h_scr.dtype)

        tail = batch % tile_cols
        if tail != 0:
            is_last = i == pl.num_programs(1) - 1
            lane = lax.broadcasted_iota(jnp.int32, h.shape, 1)
            h = jnp.where(jnp.logical_or(jnp.logical_not(is_last),
                                         lane < tail), h, 0.0)

        @pl.when(i == 0)
        def _():
            acc_ref[...] = jnp.zeros_like(acc_ref)

        acc_ref[0:HIDDEN, 0:1] = acc_ref[0:HIDDEN, 0:1] + \
            jnp.sum(h, axis=1, keepdims=True)
        acc_ref[0:HIDDEN, 1:2] = acc_ref[0:HIDDEN, 1:2] + \
            jnp.sum(h * h, axis=1, keepdims=True)

    @pl.when(phase == 1)
    def _apply_phase():
        sums = acc_ref[0:HIDDEN, 0:1]
        sqs = acc_ref[0:HIDDEN, 1:2]
        mean = sums / batch
        var = jnp.maximum(sqs / batch - mean * mean, 0.0)
        gamma = jnp.transpose(gamma_ref[...])
        beta = jnp.transpose(beta_ref[...])
        scale = gamma * lax.rsqrt(var + BN_EPS)
        shift = beta - mean * scale
        w2 = w2_ref[...]
        w2s = (w2 * scale).astype(jnp.bfloat16)
        b2p = jnp.transpose(b2_ref[...] +
                            lax.dot_general(shift, w2, (((0,), (0,)), ((), ())),
                                            preferred_element_type=jnp.float32))
        h = h_scr[i]
        logits = lax.dot_general(w2s, h, (((0,), (0,)), ((), ())),
                                 preferred_element_type=jnp.float32) + b2p
        m = jnp.max(logits, axis=0, keepdims=True)
        e = jnp.exp(logits - m)
        denom = jnp.sum(e, axis=0, keepdims=True)
        o_ref[...] = (e / denom).astype(o_ref.dtype)


def kernel(noise, w1, b1, gamma, beta, w2, b2, *, block_cols=131072):
    B = noise.shape[0]
    tb = max(LANE, min(_round_up(block_cols, LANE), _round_up(B, LANE)))
    bp = _round_up(B, tb)
    nbt = bp // tb

    xt = jnp.pad(jnp.asarray(noise, jnp.float32), ((0, bp - B), (0, 0))).T

    f32 = jnp.float32
    w1f = jnp.asarray(w1, f32)
    b1f = jnp.asarray(b1, f32).reshape(1, HIDDEN)
    gammaf = jnp.asarray(gamma, f32).reshape(1, HIDDEN)
    betaf = jnp.asarray(beta, f32).reshape(1, HIDDEN)
    w2f = jnp.asarray(w2, f32)
    b2f = jnp.asarray(b2, f32).reshape(1, OUT)

    def whole(shape):
        return pl.BlockSpec(shape, lambda p_, i: tuple(0 for _ in shape))

    out_t = pl.pallas_call(
        functools.partial(_fused_kernel, batch=B, tile_cols=tb),
        out_shape=jax.ShapeDtypeStruct((OUT, bp), jnp.float32),
        grid=(2, nbt),
        in_specs=[
            pl.BlockSpec((NUM_TOPICS, tb), lambda p_, i: (0, i * (1 - p_))),
            whole((NUM_TOPICS, HIDDEN)),
            whole((1, HIDDEN)),
            whole((1, HIDDEN)),
            whole((1, HIDDEN)),
            whole((HIDDEN, OUT)),
            whole((1, OUT)),
        ],
        out_specs=pl.BlockSpec((OUT, tb), lambda p_, i: (0, i * p_)),
        scratch_shapes=[
            pltpu.VMEM((nbt, HIDDEN, tb), jnp.bfloat16),
            pltpu.VMEM((HIDDEN + 6, LANE), jnp.float32),
        ],
        cost_estimate=pl.CostEstimate(
            flops=2 * bp * NUM_TOPICS * HIDDEN + 2 * bp * HIDDEN * OUT
                  + 18 * bp * HIDDEN,
            transcendentals=bp * OUT,
            bytes_accessed=4 * (NUM_TOPICS * bp + OUT * bp + 300)),
        compiler_params=pltpu.CompilerParams(
            dimension_semantics=("arbitrary", "arbitrary"),
            vmem_limit_bytes=64 * 1024 * 1024,
        ),
    )(xt, w1f, b1f, gammaf, betaf, w2f, b2f)

    return out_t.T[:B]
